# baseline (speedup 1.0000x reference)
import functools

import jax
import jax.numpy as jnp
from jax import lax
from jax.experimental import pallas as pl
from jax.experimental.pallas import tpu as pltpu
from jax.experimental.pallas import tpu_sc as plsc

B, H, D = 16384, 200, 64
N = B * H
NC, NS = 2, 16
NW = NC * NS
ROWS_PER_W = N // NW
IDX_MINOR = 128
CHUNK = 512
STREAMS = CHUNK // IDX_MINOR
N_CHUNKS = ROWS_PER_W // CHUNK
IDX_ROWS_W = ROWS_PER_W // IDX_MINOR

_mesh = plsc.VectorSubcoreMesh(core_axis_name="c", subcore_axis_name="s")


@functools.partial(
    pl.kernel,
    mesh=_mesh,
    out_type=jax.ShapeDtypeStruct((N, D), jnp.float32),
    compiler_params=pltpu.CompilerParams(use_tc_tiling_on_sc=False),
    scratch_types=[
        pltpu.VMEM((STREAMS, IDX_MINOR), jnp.int32),
        pltpu.VMEM((CHUNK, D), jnp.float32),
        pltpu.SemaphoreType.DMA,
    ],
)
def _gather(idx_hbm, table_hbm, out_hbm, idx_v, rows_v, gsem):
    wid = lax.axis_index("s") * NC + lax.axis_index("c")
    idx_row0 = wid * IDX_ROWS_W
    out_row0 = wid * ROWS_PER_W

    def step(c, carry):
        pltpu.sync_copy(idx_hbm.at[pl.ds(idx_row0 + c * STREAMS, STREAMS)],
                        idx_v)
        copies = [
            pltpu.async_copy(table_hbm.at[idx_v.at[j]],
                             rows_v.at[pl.ds(j * IDX_MINOR, IDX_MINOR)],
                             gsem)
            for j in range(STREAMS)
        ]
        for cp in copies:
            cp.wait()
        pltpu.sync_copy(rows_v, out_hbm.at[pl.ds(out_row0 + c * CHUNK, CHUNK)])
        return carry

    lax.fori_loop(0, N_CHUNKS, step, 0)


def kernel(x, emb):
    idx = x.reshape(N // IDX_MINOR, IDX_MINOR).astype(jnp.int32)
    out = _gather(idx, emb)
    return out.reshape(B, H, D)


# speedup vs baseline: 1.0734x; 1.0734x over previous
import functools

import jax
import jax.numpy as jnp
from jax import lax
from jax.experimental import pallas as pl
from jax.experimental.pallas import tpu as pltpu
from jax.experimental.pallas import tpu_sc as plsc

B, H, D = 16384, 200, 64
N = B * H
NC, NS = 2, 16
NW = NC * NS
ROWS_PER_W = N // NW
IDX_MINOR = 128
CHUNK = 512
STREAMS = CHUNK // IDX_MINOR
N_CHUNKS = ROWS_PER_W // CHUNK
IDX_ROWS_W = ROWS_PER_W // IDX_MINOR
G2 = N_CHUNKS // 2

_mesh = plsc.VectorSubcoreMesh(core_axis_name="c", subcore_axis_name="s")


@functools.partial(
    pl.kernel,
    mesh=_mesh,
    out_type=jax.ShapeDtypeStruct((N, D), jnp.float32),
    compiler_params=pltpu.CompilerParams(use_tc_tiling_on_sc=False),
    scratch_types=[
        pltpu.VMEM((STREAMS, IDX_MINOR), jnp.int32),
        pltpu.VMEM((STREAMS, IDX_MINOR), jnp.int32),
        pltpu.VMEM((CHUNK, D), jnp.float32),
        pltpu.VMEM((CHUNK, D), jnp.float32),
        pltpu.SemaphoreType.DMA,
        pltpu.SemaphoreType.DMA,
        pltpu.SemaphoreType.DMA,
        pltpu.SemaphoreType.DMA,
        pltpu.SemaphoreType.DMA,
        pltpu.SemaphoreType.DMA,
    ],
)
def _gather(idx_hbm, table_hbm, out_hbm,
            idx0, idx1, rows0, rows1,
            gsem0, gsem1, osem0, osem1, isem0, isem1):
    wid = lax.axis_index("s") * NC + lax.axis_index("c")
    idx_row0 = wid * IDX_ROWS_W
    out_row0 = wid * ROWS_PER_W

    idx_v = (idx0, idx1)
    rows_v = (rows0, rows1)
    gsem = (gsem0, gsem1)
    osem = (osem0, osem1)
    isem = (isem0, isem1)

    def idx_load(b, c):
        return pltpu.make_async_copy(
            idx_hbm.at[pl.ds(idx_row0 + c * STREAMS, STREAMS)],
            idx_v[b], isem[b])

    def g_copy(b, j):
        return pltpu.make_async_copy(
            table_hbm.at[idx_v[b].at[j]],
            rows_v[b].at[pl.ds(j * IDX_MINOR, IDX_MINOR)], gsem[b])

    def o_copy(b, c):
        return pltpu.make_async_copy(
            rows_v[b], out_hbm.at[pl.ds(out_row0 + c * CHUNK, CHUNK)],
            osem[b])

    idx_load(0, 0).start()
    idx_load(1, 1).start()
    idx_load(0, 0).wait()
    for j in range(STREAMS):
        g_copy(0, j).start()

    def step(g, carry):
        for b in range(2):
            c = 2 * g + b
            nb = 1 - b
            for j in range(STREAMS):
                g_copy(b, j).wait()
            o_copy(b, c).start()

            @pl.when(g < G2 - 1)
            def _():
                idx_load(b, c + 2).start()

            if b == 0:

                @pl.when(g > 0)
                def _():
                    o_copy(nb, c - 1).wait()

                idx_load(nb, c + 1).wait()
                for j in range(STREAMS):
                    g_copy(nb, j).start()
            else:
                o_copy(nb, c - 1).wait()

                @pl.when(g < G2 - 1)
                def _():
                    idx_load(nb, c + 1).wait()
                    for j in range(STREAMS):
                        g_copy(nb, j).start()
        return carry

    lax.fori_loop(0, G2, step, 0)
    o_copy(1, N_CHUNKS - 1).wait()


def kernel(x, emb):
    idx = x.reshape(N // IDX_MINOR, IDX_MINOR).astype(jnp.int32)
    out = _gather(idx, emb)
    return out.reshape(B, H, D)
